# initial kernel scaffold (unmeasured)
import jax
import jax.numpy as jnp
from jax import lax
from jax.experimental import pallas as pl
from jax.experimental.pallas import tpu as pltpu


def kernel(
    x,
):
    def body(*refs):
        pass

    out_shape = jax.ShapeDtypeStruct(..., jnp.float32)
    return pl.pallas_call(body, out_shape=out_shape)(...)


# baseline (device time: 53384 ns/iter reference)
import jax
import jax.numpy as jnp
from jax import lax
from jax.experimental import pallas as pl
from jax.experimental.pallas import tpu as pltpu

N_DEV = 8
CH = 1024


def kernel(x):
    m, n = x.shape
    n_chunks = m // CH

    def body(x_ref, out_hbm, halo_ref, chunk_out, out_sems, send_sems, recv_sems):
        my = lax.axis_index("i")
        left = (my - 1) % N_DEV
        right = (my + 1) % N_DEV

        barrier_sem = pltpu.get_barrier_semaphore()
        for nbr in (left, right):
            pl.semaphore_signal(
                barrier_sem, inc=1,
                device_id=(nbr,), device_id_type=pl.DeviceIdType.MESH,
            )
        pl.semaphore_wait(barrier_sem, 2)

        send_left = pltpu.make_async_remote_copy(
            src_ref=x_ref.at[pl.ds(0, 1)],
            dst_ref=halo_ref.at[1],
            send_sem=send_sems.at[0],
            recv_sem=recv_sems.at[1],
            device_id=(left,),
            device_id_type=pl.DeviceIdType.MESH,
        )
        send_right = pltpu.make_async_remote_copy(
            src_ref=x_ref.at[pl.ds(m - 1, 1)],
            dst_ref=halo_ref.at[0],
            send_sem=send_sems.at[1],
            recv_sem=recv_sems.at[0],
            device_id=(right,),
            device_id_type=pl.DeviceIdType.MESH,
        )
        send_left.start()
        send_right.start()

        order = list(range(1, n_chunks - 1)) + [0, n_chunks - 1]
        dmas = {}
        for pos, k in enumerate(order):
            slot = pos % 2
            if pos >= 2:
                dmas[pos - 2].wait()
            lo = k * CH
            if k == 0:
                send_left.wait()
                send_right.wait()
                chunk_out[slot, pl.ds(1, CH - 1), :] = (
                    0.25 * x_ref[pl.ds(0, CH - 1), :]
                    + 0.5 * x_ref[pl.ds(1, CH - 1), :]
                    + 0.25 * x_ref[pl.ds(2, CH - 1), :]
                )
                row0 = (
                    0.25 * halo_ref[0, :, :]
                    + 0.5 * x_ref[pl.ds(0, 1), :]
                    + 0.25 * x_ref[pl.ds(1, 1), :]
                )
                chunk_out[slot, pl.ds(0, 1), :] = jnp.where(
                    my == 0, x_ref[pl.ds(0, 1), :], row0
                )
            elif k == n_chunks - 1:
                chunk_out[slot, pl.ds(0, CH - 1), :] = (
                    0.25 * x_ref[pl.ds(lo - 1, CH - 1), :]
                    + 0.5 * x_ref[pl.ds(lo, CH - 1), :]
                    + 0.25 * x_ref[pl.ds(lo + 1, CH - 1), :]
                )
                rowl = (
                    0.25 * x_ref[pl.ds(m - 2, 1), :]
                    + 0.5 * x_ref[pl.ds(m - 1, 1), :]
                    + 0.25 * halo_ref[1, :, :]
                )
                chunk_out[slot, pl.ds(CH - 1, 1), :] = jnp.where(
                    my == N_DEV - 1, x_ref[pl.ds(m - 1, 1), :], rowl
                )
            else:
                chunk_out[slot, :, :] = (
                    0.25 * x_ref[pl.ds(lo - 1, CH), :]
                    + 0.5 * x_ref[pl.ds(lo, CH), :]
                    + 0.25 * x_ref[pl.ds(lo + 1, CH), :]
                )
            dma = pltpu.make_async_copy(
                chunk_out.at[slot], out_hbm.at[pl.ds(lo, CH)], out_sems.at[slot]
            )
            dma.start()
            dmas[pos] = dma
        dmas[len(order) - 2].wait()
        dmas[len(order) - 1].wait()

    return pl.pallas_call(
        body,
        out_shape=jax.ShapeDtypeStruct((m, n), x.dtype),
        in_specs=[pl.BlockSpec(memory_space=pltpu.VMEM)],
        out_specs=pl.BlockSpec(memory_space=pl.ANY),
        scratch_shapes=[
            pltpu.VMEM((2, 1, n), x.dtype),
            pltpu.VMEM((2, CH, n), x.dtype),
            pltpu.SemaphoreType.DMA((2,)),
            pltpu.SemaphoreType.DMA((2,)),
            pltpu.SemaphoreType.DMA((2,)),
        ],
        compiler_params=pltpu.CompilerParams(
            collective_id=0,
            vmem_limit_bytes=60 * 1024 * 1024,
        ),
    )(x)


# device time: 49015 ns/iter; 1.0891x vs baseline; 1.0891x over previous
import jax
import jax.numpy as jnp
from jax import lax
from jax.experimental import pallas as pl
from jax.experimental.pallas import tpu as pltpu

N_DEV = 8
CH = 1024
HB = 8


def kernel(x):
    m, n = x.shape
    n_chunks = m // CH
    order = list(range(1, n_chunks - 1)) + [0, n_chunks - 1]
    P = len(order)

    def body(x_hbm, out_hbm, halo_ref, cin, cout,
             in_sems, out_sems, send_sems, recv_sems):
        my = lax.axis_index("i")
        left = (my - 1) % N_DEV
        right = (my + 1) % N_DEV

        barrier_sem = pltpu.get_barrier_semaphore()
        for nbr in (left, right):
            pl.semaphore_signal(
                barrier_sem, inc=1,
                device_id=(nbr,), device_id_type=pl.DeviceIdType.MESH,
            )
        pl.semaphore_wait(barrier_sem, 2)

        send_left = pltpu.make_async_remote_copy(
            src_ref=x_hbm.at[pl.ds(0, HB)],
            dst_ref=halo_ref.at[1],
            send_sem=send_sems.at[0],
            recv_sem=recv_sems.at[1],
            device_id=(left,),
            device_id_type=pl.DeviceIdType.MESH,
        )
        send_right = pltpu.make_async_remote_copy(
            src_ref=x_hbm.at[pl.ds(m - HB, HB)],
            dst_ref=halo_ref.at[0],
            send_sem=send_sems.at[1],
            recv_sem=recv_sems.at[0],
            device_id=(right,),
            device_id_type=pl.DeviceIdType.MESH,
        )
        send_left.start()
        send_right.start()

        def make_load(pos):
            k = order[pos]
            lo = k * CH
            s = pos % 3
            if k == 0:
                return pltpu.make_async_copy(
                    x_hbm.at[pl.ds(0, CH + HB)],
                    cin.at[s, pl.ds(HB, CH + HB)],
                    in_sems.at[s],
                )
            if k == n_chunks - 1:
                return pltpu.make_async_copy(
                    x_hbm.at[pl.ds(lo - HB, CH + HB)],
                    cin.at[s, pl.ds(0, CH + HB)],
                    in_sems.at[s],
                )
            return pltpu.make_async_copy(
                x_hbm.at[pl.ds(lo - HB, CH + 2 * HB)],
                cin.at[s], in_sems.at[s],
            )

        loads = {}
        for p in range(min(2, P)):
            loads[p] = make_load(p)
            loads[p].start()

        stores = {}
        for pos, k in enumerate(order):
            s = pos % 3
            t = pos % 2
            if pos + 2 < P:
                loads[pos + 2] = make_load(pos + 2)
                loads[pos + 2].start()
            loads[pos].wait()
            if pos >= 2:
                stores[pos - 2].wait()

            lo = k * CH
            if k == 0:
                send_left.wait()
                send_right.wait()
                cout[t, pl.ds(1, CH - 1), :] = (
                    0.25 * cin[s, pl.ds(HB, CH - 1), :]
                    + 0.5 * cin[s, pl.ds(HB + 1, CH - 1), :]
                    + 0.25 * cin[s, pl.ds(HB + 2, CH - 1), :]
                )
                row0 = (
                    0.25 * halo_ref[0, pl.ds(HB - 1, 1), :]
                    + 0.5 * cin[s, pl.ds(HB, 1), :]
                    + 0.25 * cin[s, pl.ds(HB + 1, 1), :]
                )
                cout[t, pl.ds(0, 1), :] = jnp.where(
                    my == 0, cin[s, pl.ds(HB, 1), :], row0
                )
            elif k == n_chunks - 1:
                cout[t, pl.ds(0, CH - 1), :] = (
                    0.25 * cin[s, pl.ds(HB - 1, CH - 1), :]
                    + 0.5 * cin[s, pl.ds(HB, CH - 1), :]
                    + 0.25 * cin[s, pl.ds(HB + 1, CH - 1), :]
                )
                rowl = (
                    0.25 * cin[s, pl.ds(CH + HB - 2, 1), :]
                    + 0.5 * cin[s, pl.ds(CH + HB - 1, 1), :]
                    + 0.25 * halo_ref[1, pl.ds(0, 1), :]
                )
                cout[t, pl.ds(CH - 1, 1), :] = jnp.where(
                    my == N_DEV - 1, cin[s, pl.ds(CH + HB - 1, 1), :], rowl
                )
            else:
                cout[t, :, :] = (
                    0.25 * cin[s, pl.ds(HB - 1, CH), :]
                    + 0.5 * cin[s, pl.ds(HB, CH), :]
                    + 0.25 * cin[s, pl.ds(HB + 1, CH), :]
                )

            stores[pos] = pltpu.make_async_copy(
                cout.at[t], out_hbm.at[pl.ds(lo, CH)], out_sems.at[t]
            )
            stores[pos].start()

        stores[P - 2].wait()
        stores[P - 1].wait()

    return pl.pallas_call(
        body,
        out_shape=jax.ShapeDtypeStruct((m, n), x.dtype),
        in_specs=[pl.BlockSpec(memory_space=pl.ANY)],
        out_specs=pl.BlockSpec(memory_space=pl.ANY),
        scratch_shapes=[
            pltpu.VMEM((2, HB, n), x.dtype),
            pltpu.VMEM((3, CH + 2 * HB, n), x.dtype),
            pltpu.VMEM((2, CH, n), x.dtype),
            pltpu.SemaphoreType.DMA((3,)),
            pltpu.SemaphoreType.DMA((2,)),
            pltpu.SemaphoreType.DMA((2,)),
            pltpu.SemaphoreType.DMA((2,)),
        ],
        compiler_params=pltpu.CompilerParams(
            collective_id=0,
            vmem_limit_bytes=60 * 1024 * 1024,
        ),
    )(x)


# device time: 48454 ns/iter; 1.1017x vs baseline; 1.0116x over previous
import jax
import jax.numpy as jnp
from jax import lax
from jax.experimental import pallas as pl
from jax.experimental.pallas import tpu as pltpu

N_DEV = 8
CH = 2048
HB = 8


def kernel(x):
    m, n = x.shape
    n_chunks = m // CH
    order = list(range(1, n_chunks - 1)) + [0, n_chunks - 1]
    P = len(order)

    def body(x_hbm, out_hbm, halo_ref, cin, cout,
             in_sems, out_sems, send_sems, recv_sems):
        my = lax.axis_index("i")
        left = (my - 1) % N_DEV
        right = (my + 1) % N_DEV

        barrier_sem = pltpu.get_barrier_semaphore()
        for nbr in (left, right):
            pl.semaphore_signal(
                barrier_sem, inc=1,
                device_id=(nbr,), device_id_type=pl.DeviceIdType.MESH,
            )
        pl.semaphore_wait(barrier_sem, 2)

        send_left = pltpu.make_async_remote_copy(
            src_ref=x_hbm.at[pl.ds(0, HB)],
            dst_ref=halo_ref.at[1],
            send_sem=send_sems.at[0],
            recv_sem=recv_sems.at[1],
            device_id=(left,),
            device_id_type=pl.DeviceIdType.MESH,
        )
        send_right = pltpu.make_async_remote_copy(
            src_ref=x_hbm.at[pl.ds(m - HB, HB)],
            dst_ref=halo_ref.at[0],
            send_sem=send_sems.at[1],
            recv_sem=recv_sems.at[0],
            device_id=(right,),
            device_id_type=pl.DeviceIdType.MESH,
        )
        send_left.start()
        send_right.start()

        def make_load(pos):
            k = order[pos]
            lo = k * CH
            s = pos % 3
            if k == 0:
                return pltpu.make_async_copy(
                    x_hbm.at[pl.ds(0, CH + HB)],
                    cin.at[s, pl.ds(HB, CH + HB)],
                    in_sems.at[s],
                )
            if k == n_chunks - 1:
                return pltpu.make_async_copy(
                    x_hbm.at[pl.ds(lo - HB, CH + HB)],
                    cin.at[s, pl.ds(0, CH + HB)],
                    in_sems.at[s],
                )
            return pltpu.make_async_copy(
                x_hbm.at[pl.ds(lo - HB, CH + 2 * HB)],
                cin.at[s], in_sems.at[s],
            )

        loads = {}
        for p in range(min(2, P)):
            loads[p] = make_load(p)
            loads[p].start()

        stores = {}
        for pos, k in enumerate(order):
            s = pos % 3
            t = pos % 2
            if pos + 2 < P:
                loads[pos + 2] = make_load(pos + 2)
                loads[pos + 2].start()
            loads[pos].wait()
            if pos >= 2:
                stores[pos - 2].wait()

            lo = k * CH
            if k == 0:
                send_left.wait()
                send_right.wait()
                cout[t, pl.ds(1, CH - 1), :] = (
                    0.5 * cin[s, pl.ds(HB + 1, CH - 1), :]
                    + 0.25 * (
                        cin[s, pl.ds(HB, CH - 1), :]
                        + cin[s, pl.ds(HB + 2, CH - 1), :]
                    )
                )
                row0 = (
                    0.25 * halo_ref[0, pl.ds(HB - 1, 1), :]
                    + 0.5 * cin[s, pl.ds(HB, 1), :]
                    + 0.25 * cin[s, pl.ds(HB + 1, 1), :]
                )
                cout[t, pl.ds(0, 1), :] = jnp.where(
                    my == 0, cin[s, pl.ds(HB, 1), :], row0
                )
            elif k == n_chunks - 1:
                cout[t, pl.ds(0, CH - 1), :] = (
                    0.5 * cin[s, pl.ds(HB, CH - 1), :]
                    + 0.25 * (
                        cin[s, pl.ds(HB - 1, CH - 1), :]
                        + cin[s, pl.ds(HB + 1, CH - 1), :]
                    )
                )
                rowl = (
                    0.25 * cin[s, pl.ds(CH + HB - 2, 1), :]
                    + 0.5 * cin[s, pl.ds(CH + HB - 1, 1), :]
                    + 0.25 * halo_ref[1, pl.ds(0, 1), :]
                )
                cout[t, pl.ds(CH - 1, 1), :] = jnp.where(
                    my == N_DEV - 1, cin[s, pl.ds(CH + HB - 1, 1), :], rowl
                )
            else:
                cout[t, :, :] = (
                    0.5 * cin[s, pl.ds(HB, CH), :]
                    + 0.25 * (
                        cin[s, pl.ds(HB - 1, CH), :]
                        + cin[s, pl.ds(HB + 1, CH), :]
                    )
                )

            stores[pos] = pltpu.make_async_copy(
                cout.at[t], out_hbm.at[pl.ds(lo, CH)], out_sems.at[t]
            )
            stores[pos].start()

        stores[P - 2].wait()
        stores[P - 1].wait()

    return pl.pallas_call(
        body,
        out_shape=jax.ShapeDtypeStruct((m, n), x.dtype),
        in_specs=[pl.BlockSpec(memory_space=pl.ANY)],
        out_specs=pl.BlockSpec(memory_space=pl.ANY),
        scratch_shapes=[
            pltpu.VMEM((2, HB, n), x.dtype),
            pltpu.VMEM((3, CH + 2 * HB, n), x.dtype),
            pltpu.VMEM((2, CH, n), x.dtype),
            pltpu.SemaphoreType.DMA((3,)),
            pltpu.SemaphoreType.DMA((2,)),
            pltpu.SemaphoreType.DMA((2,)),
            pltpu.SemaphoreType.DMA((2,)),
        ],
        compiler_params=pltpu.CompilerParams(
            collective_id=0,
            vmem_limit_bytes=60 * 1024 * 1024,
        ),
    )(x)
